# baseline (speedup 1.0000x reference)
import jax
import jax.numpy as jnp
from jax import lax
from jax.experimental import pallas as pl
from jax.experimental.pallas import tpu as pltpu
from jax.experimental.pallas import tpu_sc as plsc

N_CLS = 3
D = 512
B = 16384

NC = 2
NS = 16
L = 16

COLS = D // NC
ROWS = B // NS
BLK = 128
NBLK = ROWS // BLK
CHUNKS = COLS // L
GROUPS = ROWS // L
SROWS = 8


def _body(feat_hbm, lab_hbm, out_hbm,
          lab_v, off_v, cnt_v, cbuf_v, tbuf_v, rbuf_v, fbuf0, fbuf1,
          sem0, sem1, shared):
  cid = lax.axis_index("c")
  sid = lax.axis_index("s")
  row0 = sid * ROWS
  col0 = cid * COLS

  iota = lax.iota(jnp.int32, L)
  zf = jnp.zeros((L,), jnp.float32)
  zi = jnp.zeros((L,), jnp.int32)

  fbufs = (fbuf0, fbuf1)
  lsems = (sem0, sem1)

  def mk_load(blk, par):
    return pltpu.make_async_copy(
        feat_hbm.at[pl.ds(row0 + blk * BLK, BLK), pl.ds(col0, COLS)],
        fbufs[par], lsems[par])

  mk_load(0, 0).start()
  mk_load(1, 1).start()

  pltpu.sync_copy(lab_hbm.at[pl.ds(row0 * N_CLS, ROWS * N_CLS)], lab_v)

  for k in range(6):
    for j in range(CHUNKS):
      cbuf_v[k, pl.ds(j * L, L)] = zf
  for k in range(N_CLS):
    cnt_v[k, pl.ds(0, L)] = zf

  iota3 = iota * 3
  one = jnp.float32(1.0)
  zero = jnp.float32(0.0)

  @plsc.parallel_loop(0, GROUPS, unroll=4)
  def _cls(g):
    base = g * (L * N_CLS)
    p0 = plsc.load_gather(lab_v, [iota3 + base])
    p1 = plsc.load_gather(lab_v, [iota3 + (base + 1)])
    p2 = plsc.load_gather(lab_v, [iota3 + (base + 2)])
    cls = jnp.where(p1 > p0, 1, 0).astype(jnp.int32)
    m01 = jnp.maximum(p0, p1)
    cls = jnp.where(p2 > m01, 2, cls)
    off_v[pl.ds(g * L, L)] = cls
    plsc.addupdate(cnt_v.at[0, pl.ds(0, L)], jnp.where(cls == 0, one, zero))
    plsc.addupdate(cnt_v.at[1, pl.ds(0, L)], jnp.where(cls == 1, one, zero))
    plsc.addupdate(cnt_v.at[2, pl.ds(0, L)], jnp.where(cls == 2, one, zero))

  def process(blk, fbuf):
    @plsc.parallel_loop(0, BLK, unroll=8)
    def _rows(r):
      cls_splat = plsc.load_gather(off_v, [zi + (blk * BLK + r)])
      for j in range(CHUNKS):
        x = fbuf[r, pl.ds(j * L, L)]
        plsc.addupdate_scatter(cbuf_v, [cls_splat, iota + j * L], x)

  for blk in range(NBLK):
    par = blk % 2
    mk_load(blk, par).wait()
    process(blk, fbufs[par])
    if blk + 2 < NBLK:
      mk_load(blk + 2, par).start()

  n0 = zf + jnp.sum(cnt_v[0, pl.ds(0, L)])
  n1 = zf + jnp.sum(cnt_v[1, pl.ds(0, L)])
  n2 = zf + jnp.sum(cnt_v[2, pl.ds(0, L)])
  for j in range(CHUNKS):
    cbuf_v[3, pl.ds(j * L, L)] = n0
    cbuf_v[4, pl.ds(j * L, L)] = n1
    cbuf_v[5, pl.ds(j * L, L)] = n2
  pltpu.sync_copy(cbuf_v, shared.at[pl.ds(sid * SROWS, SROWS)])
  plsc.subcore_barrier()

  for d in (8, 4, 2, 1):
    @pl.when(sid < d)
    def _step(d=d):
      pltpu.sync_copy(shared.at[pl.ds((sid + d) * SROWS, SROWS)], tbuf_v)
      for k in range(6):
        for j in range(CHUNKS):
          cbuf_v[k, pl.ds(j * L, L)] = (
              cbuf_v[k, pl.ds(j * L, L)] + tbuf_v[k, pl.ds(j * L, L)])
      if d > 1:
        pltpu.sync_copy(cbuf_v, shared.at[pl.ds(sid * SROWS, SROWS)])
    plsc.subcore_barrier()

  @pl.when(sid == 0)
  def _finalize():
    ones = jnp.ones((L,), jnp.float32)
    for k in range(N_CLS):
      for j in range(CHUNKS):
        cnt = cbuf_v[3 + k, pl.ds(j * L, L)]
        safe = jnp.where(cnt > 0, cnt, ones)
        scale = jnp.where(cnt > 0, ones / safe, ones)
        cbuf_v[k, pl.ds(j * L, L)] = cbuf_v[k, pl.ds(j * L, L)] * scale
    pltpu.sync_copy(cbuf_v.at[pl.ds(0, N_CLS)],
                    out_hbm.at[:, pl.ds(col0, COLS)])


@jax.jit
def kernel(features, pseudo_labels):
  mesh = plsc.VectorSubcoreMesh(core_axis_name="c", subcore_axis_name="s")
  run = pl.kernel(
      _body,
      out_type=jax.ShapeDtypeStruct((N_CLS, D), jnp.float32),
      mesh=mesh,
      compiler_params=pltpu.CompilerParams(needs_layout_passes=False),
      scratch_types=[
          pltpu.VMEM((ROWS * N_CLS,), jnp.float32),
          pltpu.VMEM((ROWS,), jnp.int32),
          pltpu.VMEM((N_CLS, L), jnp.float32),
          pltpu.VMEM((SROWS, COLS), jnp.float32),
          pltpu.VMEM((SROWS, COLS), jnp.float32),
          pltpu.VMEM((SROWS, COLS), jnp.float32),
          pltpu.VMEM((BLK, COLS), jnp.float32),
          pltpu.VMEM((BLK, COLS), jnp.float32),
          pltpu.SemaphoreType.DMA,
          pltpu.SemaphoreType.DMA,
          pltpu.VMEM_SHARED((NS * SROWS, COLS), jnp.float32),
      ],
  )
  return run(features, pseudo_labels.reshape(-1))


# speedup vs baseline: 1.1011x; 1.1011x over previous
import jax
import jax.numpy as jnp
from jax import lax
from jax.experimental import pallas as pl
from jax.experimental.pallas import tpu as pltpu
from jax.experimental.pallas import tpu_sc as plsc

N_CLS = 3
D = 512
B = 16384

NC = 2
NS = 16
L = 16

COLS = D // NC
ROWS = B // NS
BLK = 128
NBLK = ROWS // BLK
CHUNKS = COLS // L
GROUPS = ROWS // L
SROWS = 8


def _body(feat_hbm, lab_hbm, out_hbm,
          lab_v, off_v, cnt_v, cbuf_v, tbuf_v, rbuf_v, fbuf0, fbuf1,
          sem0, sem1, shared):
  cid = lax.axis_index("c")
  sid = lax.axis_index("s")
  row0 = sid * ROWS
  col0 = cid * COLS

  iota = lax.iota(jnp.int32, L)
  zf = jnp.zeros((L,), jnp.float32)
  zi = jnp.zeros((L,), jnp.int32)

  fbufs = (fbuf0, fbuf1)
  lsems = (sem0, sem1)

  def mk_load(blk, par):
    return pltpu.make_async_copy(
        feat_hbm.at[pl.ds(row0 + blk * BLK, BLK), pl.ds(col0, COLS)],
        fbufs[par], lsems[par])

  mk_load(0, 0).start()
  mk_load(1, 1).start()

  pltpu.sync_copy(lab_hbm.at[pl.ds(row0 * N_CLS, ROWS * N_CLS)], lab_v)

  for k in range(6):
    for j in range(CHUNKS):
      cbuf_v[k, pl.ds(j * L, L)] = zf
  for k in range(N_CLS):
    cnt_v[k, pl.ds(0, L)] = zf

  iota3 = iota * 3
  one = jnp.float32(1.0)
  zero = jnp.float32(0.0)

  @plsc.parallel_loop(0, GROUPS, unroll=4)
  def _cls(g):
    base = g * (L * N_CLS)
    p0 = plsc.load_gather(lab_v, [iota3 + base])
    p1 = plsc.load_gather(lab_v, [iota3 + (base + 1)])
    p2 = plsc.load_gather(lab_v, [iota3 + (base + 2)])
    cls = jnp.where(p1 > p0, 1, 0).astype(jnp.int32)
    m01 = jnp.maximum(p0, p1)
    cls = jnp.where(p2 > m01, 2, cls)
    off_v[pl.ds(g * L, L)] = cls
    plsc.addupdate(cnt_v.at[0, pl.ds(0, L)], jnp.where(cls == 0, one, zero))
    plsc.addupdate(cnt_v.at[1, pl.ds(0, L)], jnp.where(cls == 1, one, zero))
    plsc.addupdate(cnt_v.at[2, pl.ds(0, L)], jnp.where(cls == 2, one, zero))

  def process(blk, fbuf):
    @plsc.parallel_loop(0, BLK, unroll=2)
    def _rows(r):
      cls_splat = plsc.load_gather(off_v, [zi + (blk * BLK + r)])
      for j in range(CHUNKS):
        x = fbuf[r, pl.ds(j * L, L)]
        plsc.addupdate_scatter(cbuf_v, [cls_splat, iota + j * L], x)

  for blk in range(NBLK):
    par = blk % 2
    mk_load(blk, par).wait()
    process(blk, fbufs[par])
    if blk + 2 < NBLK:
      mk_load(blk + 2, par).start()

  n0 = zf + jnp.sum(cnt_v[0, pl.ds(0, L)])
  n1 = zf + jnp.sum(cnt_v[1, pl.ds(0, L)])
  n2 = zf + jnp.sum(cnt_v[2, pl.ds(0, L)])
  for j in range(CHUNKS):
    cbuf_v[3, pl.ds(j * L, L)] = n0
    cbuf_v[4, pl.ds(j * L, L)] = n1
    cbuf_v[5, pl.ds(j * L, L)] = n2
  pltpu.sync_copy(cbuf_v, shared.at[pl.ds(sid * SROWS, SROWS)])
  plsc.subcore_barrier()

  for d in (8, 4, 2, 1):
    @pl.when(sid < d)
    def _step(d=d):
      pltpu.sync_copy(shared.at[pl.ds((sid + d) * SROWS, SROWS)], tbuf_v)
      for k in range(6):
        for j in range(CHUNKS):
          cbuf_v[k, pl.ds(j * L, L)] = (
              cbuf_v[k, pl.ds(j * L, L)] + tbuf_v[k, pl.ds(j * L, L)])
      if d > 1:
        pltpu.sync_copy(cbuf_v, shared.at[pl.ds(sid * SROWS, SROWS)])
    plsc.subcore_barrier()

  @pl.when(sid == 0)
  def _finalize():
    ones = jnp.ones((L,), jnp.float32)
    for k in range(N_CLS):
      for j in range(CHUNKS):
        cnt = cbuf_v[3 + k, pl.ds(j * L, L)]
        safe = jnp.where(cnt > 0, cnt, ones)
        scale = jnp.where(cnt > 0, ones / safe, ones)
        cbuf_v[k, pl.ds(j * L, L)] = cbuf_v[k, pl.ds(j * L, L)] * scale
    pltpu.sync_copy(cbuf_v.at[pl.ds(0, N_CLS)],
                    out_hbm.at[:, pl.ds(col0, COLS)])


@jax.jit
def kernel(features, pseudo_labels):
  mesh = plsc.VectorSubcoreMesh(core_axis_name="c", subcore_axis_name="s")
  run = pl.kernel(
      _body,
      out_type=jax.ShapeDtypeStruct((N_CLS, D), jnp.float32),
      mesh=mesh,
      compiler_params=pltpu.CompilerParams(needs_layout_passes=False),
      scratch_types=[
          pltpu.VMEM((ROWS * N_CLS,), jnp.float32),
          pltpu.VMEM((ROWS,), jnp.int32),
          pltpu.VMEM((N_CLS, L), jnp.float32),
          pltpu.VMEM((SROWS, COLS), jnp.float32),
          pltpu.VMEM((SROWS, COLS), jnp.float32),
          pltpu.VMEM((SROWS, COLS), jnp.float32),
          pltpu.VMEM((BLK, COLS), jnp.float32),
          pltpu.VMEM((BLK, COLS), jnp.float32),
          pltpu.SemaphoreType.DMA,
          pltpu.SemaphoreType.DMA,
          pltpu.VMEM_SHARED((NS * SROWS, COLS), jnp.float32),
      ],
  )
  return run(features, pseudo_labels.reshape(-1))


# speedup vs baseline: 1.1179x; 1.0153x over previous
import jax
import jax.numpy as jnp
from jax import lax
from jax.experimental import pallas as pl
from jax.experimental.pallas import tpu as pltpu
from jax.experimental.pallas import tpu_sc as plsc

N_CLS = 3
D = 512
B = 16384

NC = 2
NS = 16
L = 16

COLS = D // NC
ROWS = B // NS
BLK = 128
NBLK = ROWS // BLK
CHUNKS = COLS // L
GROUPS = ROWS // L
SROWS = 8


def _body(feat_hbm, lab_hbm, out_hbm,
          lab_v, off_v, cnt_v, cbuf_v, tbuf_v, rbuf_v, fbuf0, fbuf1,
          sem0, sem1, shared):
  cid = lax.axis_index("c")
  sid = lax.axis_index("s")
  row0 = sid * ROWS
  col0 = cid * COLS

  iota = lax.iota(jnp.int32, L)
  zf = jnp.zeros((L,), jnp.float32)
  zi = jnp.zeros((L,), jnp.int32)

  fbufs = (fbuf0, fbuf1)
  lsems = (sem0, sem1)

  def mk_load(blk, par):
    return pltpu.make_async_copy(
        feat_hbm.at[pl.ds(row0 + blk * BLK, BLK), pl.ds(col0, COLS)],
        fbufs[par], lsems[par])

  mk_load(0, 0).start()
  mk_load(1, 1).start()

  pltpu.sync_copy(lab_hbm.at[pl.ds(row0 * N_CLS, ROWS * N_CLS)], lab_v)

  for k in range(6):
    for j in range(CHUNKS):
      cbuf_v[k, pl.ds(j * L, L)] = zf
  for k in range(N_CLS):
    cnt_v[k, pl.ds(0, L)] = zf

  iota3 = iota * 3
  one = jnp.float32(1.0)
  zero = jnp.float32(0.0)

  @plsc.parallel_loop(0, GROUPS, unroll=4)
  def _cls(g):
    base = g * (L * N_CLS)
    p0 = plsc.load_gather(lab_v, [iota3 + base])
    p1 = plsc.load_gather(lab_v, [iota3 + (base + 1)])
    p2 = plsc.load_gather(lab_v, [iota3 + (base + 2)])
    cls = jnp.where(p1 > p0, 1, 0).astype(jnp.int32)
    m01 = jnp.maximum(p0, p1)
    cls = jnp.where(p2 > m01, 2, cls)
    off_v[pl.ds(g * L, L)] = cls
    plsc.addupdate(cnt_v.at[0, pl.ds(0, L)], jnp.where(cls == 0, one, zero))
    plsc.addupdate(cnt_v.at[1, pl.ds(0, L)], jnp.where(cls == 1, one, zero))
    plsc.addupdate(cnt_v.at[2, pl.ds(0, L)], jnp.where(cls == 2, one, zero))

  def process(blk, fbuf):
    @plsc.parallel_loop(0, BLK, unroll=1)
    def _rows(r):
      cls_splat = plsc.load_gather(off_v, [zi + (blk * BLK + r)])
      for j in range(CHUNKS):
        x = fbuf[r, pl.ds(j * L, L)]
        plsc.addupdate_scatter(cbuf_v, [cls_splat, iota + j * L], x)

  for blk in range(NBLK):
    par = blk % 2
    mk_load(blk, par).wait()
    process(blk, fbufs[par])
    if blk + 2 < NBLK:
      mk_load(blk + 2, par).start()

  n0 = zf + jnp.sum(cnt_v[0, pl.ds(0, L)])
  n1 = zf + jnp.sum(cnt_v[1, pl.ds(0, L)])
  n2 = zf + jnp.sum(cnt_v[2, pl.ds(0, L)])
  for j in range(CHUNKS):
    cbuf_v[3, pl.ds(j * L, L)] = n0
    cbuf_v[4, pl.ds(j * L, L)] = n1
    cbuf_v[5, pl.ds(j * L, L)] = n2
  pltpu.sync_copy(cbuf_v, shared.at[pl.ds(sid * SROWS, SROWS)])
  plsc.subcore_barrier()

  for d in (8, 4, 2, 1):
    @pl.when(sid < d)
    def _step(d=d):
      pltpu.sync_copy(shared.at[pl.ds((sid + d) * SROWS, SROWS)], tbuf_v)
      for k in range(6):
        for j in range(CHUNKS):
          cbuf_v[k, pl.ds(j * L, L)] = (
              cbuf_v[k, pl.ds(j * L, L)] + tbuf_v[k, pl.ds(j * L, L)])
      if d > 1:
        pltpu.sync_copy(cbuf_v, shared.at[pl.ds(sid * SROWS, SROWS)])
    plsc.subcore_barrier()

  @pl.when(sid == 0)
  def _finalize():
    ones = jnp.ones((L,), jnp.float32)
    for k in range(N_CLS):
      for j in range(CHUNKS):
        cnt = cbuf_v[3 + k, pl.ds(j * L, L)]
        safe = jnp.where(cnt > 0, cnt, ones)
        scale = jnp.where(cnt > 0, ones / safe, ones)
        cbuf_v[k, pl.ds(j * L, L)] = cbuf_v[k, pl.ds(j * L, L)] * scale
    pltpu.sync_copy(cbuf_v.at[pl.ds(0, N_CLS)],
                    out_hbm.at[:, pl.ds(col0, COLS)])


@jax.jit
def kernel(features, pseudo_labels):
  mesh = plsc.VectorSubcoreMesh(core_axis_name="c", subcore_axis_name="s")
  run = pl.kernel(
      _body,
      out_type=jax.ShapeDtypeStruct((N_CLS, D), jnp.float32),
      mesh=mesh,
      compiler_params=pltpu.CompilerParams(needs_layout_passes=False),
      scratch_types=[
          pltpu.VMEM((ROWS * N_CLS,), jnp.float32),
          pltpu.VMEM((ROWS,), jnp.int32),
          pltpu.VMEM((N_CLS, L), jnp.float32),
          pltpu.VMEM((SROWS, COLS), jnp.float32),
          pltpu.VMEM((SROWS, COLS), jnp.float32),
          pltpu.VMEM((SROWS, COLS), jnp.float32),
          pltpu.VMEM((BLK, COLS), jnp.float32),
          pltpu.VMEM((BLK, COLS), jnp.float32),
          pltpu.SemaphoreType.DMA,
          pltpu.SemaphoreType.DMA,
          pltpu.VMEM_SHARED((NS * SROWS, COLS), jnp.float32),
      ],
  )
  return run(features, pseudo_labels.reshape(-1))


# speedup vs baseline: 1.1202x; 1.0021x over previous
import jax
import jax.numpy as jnp
from jax import lax
from jax.experimental import pallas as pl
from jax.experimental.pallas import tpu as pltpu
from jax.experimental.pallas import tpu_sc as plsc

N_CLS = 3
D = 512
B = 16384

NC = 2
NS = 16
L = 16

COLS = D // NC
ROWS = B // NS
BLK = 128
NBLK = ROWS // BLK
CHUNKS = COLS // L
GROUPS = ROWS // L
SROWS = 8


def _body(feat_hbm, lab_hbm, out_hbm,
          lab_v, off_v, cnt_v, cbuf_v, tbuf_v, rbuf_v, fbuf0, fbuf1,
          sem0, sem1, shared):
  cid = lax.axis_index("c")
  sid = lax.axis_index("s")
  row0 = sid * ROWS
  col0 = cid * COLS

  iota = lax.iota(jnp.int32, L)
  zf = jnp.zeros((L,), jnp.float32)
  zi = jnp.zeros((L,), jnp.int32)

  fbufs = (fbuf0, fbuf1)
  lsems = (sem0, sem1)

  def mk_load(blk, par):
    return pltpu.make_async_copy(
        feat_hbm.at[pl.ds(row0 + blk * BLK, BLK), pl.ds(col0, COLS)],
        fbufs[par], lsems[par])

  mk_load(0, 0).start()
  mk_load(1, 1).start()

  pltpu.sync_copy(lab_hbm.at[pl.ds(row0 * N_CLS, ROWS * N_CLS)], lab_v)

  for k in range(6):
    for j in range(CHUNKS):
      cbuf_v[k, pl.ds(j * L, L)] = zf
  for k in range(N_CLS):
    cnt_v[k, pl.ds(0, L)] = zf

  iota3 = iota * 3
  one = jnp.float32(1.0)
  zero = jnp.float32(0.0)

  @plsc.parallel_loop(0, GROUPS, unroll=1)
  def _cls(g):
    base = g * (L * N_CLS)
    p0 = plsc.load_gather(lab_v, [iota3 + base])
    p1 = plsc.load_gather(lab_v, [iota3 + (base + 1)])
    p2 = plsc.load_gather(lab_v, [iota3 + (base + 2)])
    cls = jnp.where(p1 > p0, 1, 0).astype(jnp.int32)
    m01 = jnp.maximum(p0, p1)
    cls = jnp.where(p2 > m01, 2, cls)
    off_v[pl.ds(g * L, L)] = cls
    plsc.addupdate(cnt_v.at[0, pl.ds(0, L)], jnp.where(cls == 0, one, zero))
    plsc.addupdate(cnt_v.at[1, pl.ds(0, L)], jnp.where(cls == 1, one, zero))
    plsc.addupdate(cnt_v.at[2, pl.ds(0, L)], jnp.where(cls == 2, one, zero))

  def process(blk, fbuf):
    @plsc.parallel_loop(0, BLK, unroll=1)
    def _rows(r):
      cls_splat = plsc.load_gather(off_v, [zi + (blk * BLK + r)])
      for j in range(CHUNKS):
        x = fbuf[r, pl.ds(j * L, L)]
        plsc.addupdate_scatter(cbuf_v, [cls_splat, iota + j * L], x)

  for blk in range(NBLK):
    par = blk % 2
    mk_load(blk, par).wait()
    process(blk, fbufs[par])
    if blk + 2 < NBLK:
      mk_load(blk + 2, par).start()

  n0 = zf + jnp.sum(cnt_v[0, pl.ds(0, L)])
  n1 = zf + jnp.sum(cnt_v[1, pl.ds(0, L)])
  n2 = zf + jnp.sum(cnt_v[2, pl.ds(0, L)])
  for j in range(CHUNKS):
    cbuf_v[3, pl.ds(j * L, L)] = n0
    cbuf_v[4, pl.ds(j * L, L)] = n1
    cbuf_v[5, pl.ds(j * L, L)] = n2
  pltpu.sync_copy(cbuf_v, shared.at[pl.ds(sid * SROWS, SROWS)])
  plsc.subcore_barrier()

  for d in (8, 4, 2, 1):
    @pl.when(sid < d)
    def _step(d=d):
      pltpu.sync_copy(shared.at[pl.ds((sid + d) * SROWS, SROWS)], tbuf_v)
      for k in range(6):
        for j in range(CHUNKS):
          cbuf_v[k, pl.ds(j * L, L)] = (
              cbuf_v[k, pl.ds(j * L, L)] + tbuf_v[k, pl.ds(j * L, L)])
      if d > 1:
        pltpu.sync_copy(cbuf_v, shared.at[pl.ds(sid * SROWS, SROWS)])
    plsc.subcore_barrier()

  @pl.when(sid == 0)
  def _finalize():
    ones = jnp.ones((L,), jnp.float32)
    for k in range(N_CLS):
      for j in range(CHUNKS):
        cnt = cbuf_v[3 + k, pl.ds(j * L, L)]
        safe = jnp.where(cnt > 0, cnt, ones)
        scale = jnp.where(cnt > 0, ones / safe, ones)
        cbuf_v[k, pl.ds(j * L, L)] = cbuf_v[k, pl.ds(j * L, L)] * scale
    pltpu.sync_copy(cbuf_v.at[pl.ds(0, N_CLS)],
                    out_hbm.at[:, pl.ds(col0, COLS)])


@jax.jit
def kernel(features, pseudo_labels):
  mesh = plsc.VectorSubcoreMesh(core_axis_name="c", subcore_axis_name="s")
  run = pl.kernel(
      _body,
      out_type=jax.ShapeDtypeStruct((N_CLS, D), jnp.float32),
      mesh=mesh,
      compiler_params=pltpu.CompilerParams(needs_layout_passes=False),
      scratch_types=[
          pltpu.VMEM((ROWS * N_CLS,), jnp.float32),
          pltpu.VMEM((ROWS,), jnp.int32),
          pltpu.VMEM((N_CLS, L), jnp.float32),
          pltpu.VMEM((SROWS, COLS), jnp.float32),
          pltpu.VMEM((SROWS, COLS), jnp.float32),
          pltpu.VMEM((SROWS, COLS), jnp.float32),
          pltpu.VMEM((BLK, COLS), jnp.float32),
          pltpu.VMEM((BLK, COLS), jnp.float32),
          pltpu.SemaphoreType.DMA,
          pltpu.SemaphoreType.DMA,
          pltpu.VMEM_SHARED((NS * SROWS, COLS), jnp.float32),
      ],
  )
  return run(features, pseudo_labels.reshape(-1))


# speedup vs baseline: 1.1212x; 1.0009x over previous
import jax
import jax.numpy as jnp
from jax import lax
from jax.experimental import pallas as pl
from jax.experimental.pallas import tpu as pltpu
from jax.experimental.pallas import tpu_sc as plsc

N_CLS = 3
D = 512
B = 16384

NC = 2
NS = 16
L = 16

COLS = D // NC
ROWS = B // NS
BLK = 128
NBLK = ROWS // BLK
CHUNKS = COLS // L
GROUPS = ROWS // L
SROWS = 8


def _body(feat_hbm, lab_hbm, out_hbm,
          lab_v, off_v, cnt_v, cbuf_v, tbuf_v, fbuf0, fbuf1,
          sem0, sem1, shared):
  cid = lax.axis_index("c")
  sid = lax.axis_index("s")
  row0 = sid * ROWS
  col0 = cid * COLS

  iota = lax.iota(jnp.int32, L)
  zf = jnp.zeros((L,), jnp.float32)
  zi = jnp.zeros((L,), jnp.int32)

  fbufs = (fbuf0, fbuf1)
  lsems = (sem0, sem1)

  def mk_load(blk, par):
    return pltpu.make_async_copy(
        feat_hbm.at[pl.ds(row0 + blk * BLK, BLK), pl.ds(col0, COLS)],
        fbufs[par], lsems[par])

  mk_load(0, 0).start()
  mk_load(1, 1).start()

  pltpu.sync_copy(lab_hbm.at[pl.ds(row0 * N_CLS, ROWS * N_CLS)], lab_v)

  for k in range(6):
    for j in range(CHUNKS):
      cbuf_v[k, pl.ds(j * L, L)] = zf
  for k in range(N_CLS):
    cnt_v[k, pl.ds(0, L)] = zf

  iota3 = iota * 3
  one = jnp.float32(1.0)
  zero = jnp.float32(0.0)

  @plsc.parallel_loop(0, GROUPS, unroll=1)
  def _cls(g):
    base = g * (L * N_CLS)
    p0 = plsc.load_gather(lab_v, [iota3 + base])
    p1 = plsc.load_gather(lab_v, [iota3 + (base + 1)])
    p2 = plsc.load_gather(lab_v, [iota3 + (base + 2)])
    cls = jnp.where(p1 > p0, 1, 0).astype(jnp.int32)
    m01 = jnp.maximum(p0, p1)
    cls = jnp.where(p2 > m01, 2, cls)
    off_v[pl.ds(g * L, L)] = cls
    plsc.addupdate(cnt_v.at[0, pl.ds(0, L)], jnp.where(cls == 0, one, zero))
    plsc.addupdate(cnt_v.at[1, pl.ds(0, L)], jnp.where(cls == 1, one, zero))
    plsc.addupdate(cnt_v.at[2, pl.ds(0, L)], jnp.where(cls == 2, one, zero))

  def process(blk, fbuf):
    @plsc.parallel_loop(0, BLK, unroll=1)
    def _rows(r):
      cls_splat = plsc.load_gather(off_v, [zi + (blk * BLK + r)])
      for j in range(CHUNKS):
        x = fbuf[r, pl.ds(j * L, L)]
        plsc.addupdate_scatter(cbuf_v, [cls_splat, iota + j * L], x)

  for blk in range(NBLK):
    par = blk % 2
    mk_load(blk, par).wait()
    process(blk, fbufs[par])
    if blk + 2 < NBLK:
      mk_load(blk + 2, par).start()

  n0 = zf + jnp.sum(cnt_v[0, pl.ds(0, L)])
  n1 = zf + jnp.sum(cnt_v[1, pl.ds(0, L)])
  n2 = zf + jnp.sum(cnt_v[2, pl.ds(0, L)])
  for j in range(CHUNKS):
    cbuf_v[3, pl.ds(j * L, L)] = n0
    cbuf_v[4, pl.ds(j * L, L)] = n1
    cbuf_v[5, pl.ds(j * L, L)] = n2
  pltpu.sync_copy(cbuf_v, shared.at[pl.ds(sid * SROWS, SROWS)])
  plsc.subcore_barrier()

  for d in (8, 4, 2, 1):
    @pl.when(sid < d)
    def _step(d=d):
      pltpu.sync_copy(shared.at[pl.ds((sid + d) * SROWS, SROWS)], tbuf_v)
      for k in range(6):
        for j in range(CHUNKS):
          cbuf_v[k, pl.ds(j * L, L)] = (
              cbuf_v[k, pl.ds(j * L, L)] + tbuf_v[k, pl.ds(j * L, L)])
      if d > 1:
        pltpu.sync_copy(cbuf_v, shared.at[pl.ds(sid * SROWS, SROWS)])
    plsc.subcore_barrier()

  @pl.when(sid == 0)
  def _finalize():
    ones = jnp.ones((L,), jnp.float32)
    for k in range(N_CLS):
      for j in range(CHUNKS):
        cnt = cbuf_v[3 + k, pl.ds(j * L, L)]
        safe = jnp.where(cnt > 0, cnt, ones)
        scale = jnp.where(cnt > 0, ones / safe, ones)
        cbuf_v[k, pl.ds(j * L, L)] = cbuf_v[k, pl.ds(j * L, L)] * scale
    pltpu.sync_copy(cbuf_v.at[pl.ds(0, N_CLS)],
                    out_hbm.at[:, pl.ds(col0, COLS)])


@jax.jit
def kernel(features, pseudo_labels):
  mesh = plsc.VectorSubcoreMesh(core_axis_name="c", subcore_axis_name="s")
  run = pl.kernel(
      _body,
      out_type=jax.ShapeDtypeStruct((N_CLS, D), jnp.float32),
      mesh=mesh,
      compiler_params=pltpu.CompilerParams(needs_layout_passes=False),
      scratch_types=[
          pltpu.VMEM((ROWS * N_CLS,), jnp.float32),
          pltpu.VMEM((ROWS,), jnp.int32),
          pltpu.VMEM((N_CLS, L), jnp.float32),
          pltpu.VMEM((SROWS, COLS), jnp.float32),
          pltpu.VMEM((SROWS, COLS), jnp.float32),
          pltpu.VMEM((BLK, COLS), jnp.float32),
          pltpu.VMEM((BLK, COLS), jnp.float32),
          pltpu.SemaphoreType.DMA,
          pltpu.SemaphoreType.DMA,
          pltpu.VMEM_SHARED((NS * SROWS, COLS), jnp.float32),
      ],
  )
  return run(features, pseudo_labels.reshape(-1))
